# baseline (speedup 1.0000x reference)
import jax
import jax.numpy as jnp
from jax.experimental import pallas as pl
from jax.experimental.pallas import tpu as pltpu

_FEAT_STRIDE = 16.0


def _anchor_pattern():
    r = jax.lax.broadcasted_iota(jnp.int32, (1152, 128), 0)
    l = jax.lax.broadcasted_iota(jnp.int32, (1152, 128), 1)
    rf = r.astype(jnp.float32)
    g = jnp.floor(rf * 0.25)
    c = rf - 4.0 * g
    n = g * 128.0 + l.astype(jnp.float32)
    k = jnp.floor((n + 0.5) * (1.0 / 9.0))
    a = n - 9.0 * k
    ri = jnp.floor((a + 0.5) * (1.0 / 3.0))
    si = a - 3.0 * ri
    ws = jnp.where(ri < 0.5, 23.0, jnp.where(ri < 1.5, 16.0, 11.0))
    hs = jnp.where(ri < 0.5, 12.0, jnp.where(ri < 1.5, 16.0, 22.0))
    sc = jnp.where(si < 0.5, 8.0, jnp.where(si < 1.5, 16.0, 32.0))
    hw = 0.5 * (ws * sc - 1.0)
    hh = 0.5 * (hs * sc - 1.0)
    base = jnp.where(c < 0.5, 7.5 - hw,
                     jnp.where(c < 1.5, 7.5 - hh,
                               jnp.where(c < 2.5, 7.5 + hw, 7.5 + hh)))
    y = jnp.floor(k * (1.0 / 64.0))
    x = k - 64.0 * y
    c_even = jnp.logical_or(c < 0.5, jnp.abs(c - 2.0) < 0.5)
    return base + _FEAT_STRIDE * jnp.where(c_even, x, y)


def _body(scores_ref, bbox_ref, fg_ref, bbox_out_ref, anc_ref, pat_ref):
    h = pl.program_id(1)

    @pl.when(jnp.logical_and(pl.program_id(0) == 0, h == 0))
    def _():
        pat_ref[...] = _anchor_pattern()

    fg_ref[...] = scores_ref[...]
    bbox_out_ref[...] = bbox_ref[...]
    anc_ref[0] = pat_ref[pl.ds(h * 576, 576), :]


def kernel(scores, bbox_deltas, im_info, cfg_key):
    B = scores.shape[0]
    A = 9
    H, W = scores.shape[2], scores.shape[3]
    K = H * W

    C = bbox_deltas.shape[1]
    fg, bbox_out, anc = pl.pallas_call(
        _body,
        grid=(B, 2),
        in_specs=[
            pl.BlockSpec((1, A, H // 2, W), lambda b, h: (b, 1, h, 0)),
            pl.BlockSpec((1, C, H // 2, W), lambda b, h: (b, 0, h, 0)),
        ],
        out_specs=[
            pl.BlockSpec((1, A, H // 2, W), lambda b, h: (b, 0, h, 0)),
            pl.BlockSpec((1, C, H // 2, W), lambda b, h: (b, 0, h, 0)),
            pl.BlockSpec((1, (K * A * 2) // 128, 128), lambda b, h: (b, h, 0)),
        ],
        out_shape=[
            jax.ShapeDtypeStruct((B, A, H, W), jnp.float32),
            jax.ShapeDtypeStruct((B, C, H, W), jnp.float32),
            jax.ShapeDtypeStruct((B, (K * A * 4) // 128, 128), jnp.float32),
        ],
        scratch_shapes=[pltpu.VMEM(((K * A * 4) // 128, 128), jnp.float32)],
        compiler_params=pltpu.CompilerParams(
            dimension_semantics=("arbitrary", "arbitrary"),
        ),
    )(scores, bbox_deltas)

    anchors = (anc.reshape(B, (K * A) // 128, 4, 128)
               .transpose(0, 1, 3, 2)
               .reshape(B, K * A, 4))
    return (fg, bbox_out, im_info, anchors)


# speedup vs baseline: 1.5435x; 1.5435x over previous
import jax
import jax.numpy as jnp
from jax.experimental import pallas as pl
from jax.experimental.pallas import tpu as pltpu

_FEAT_STRIDE = 16.0


def _anchor_pattern():
    r = jax.lax.broadcasted_iota(jnp.int32, (1152, 128), 0)
    l = jax.lax.broadcasted_iota(jnp.int32, (1152, 128), 1)
    rf = r.astype(jnp.float32)
    g = jnp.floor(rf * 0.25)
    c = rf - 4.0 * g
    n = g * 128.0 + l.astype(jnp.float32)
    k = jnp.floor((n + 0.5) * (1.0 / 9.0))
    a = n - 9.0 * k
    ri = jnp.floor((a + 0.5) * (1.0 / 3.0))
    si = a - 3.0 * ri
    ws = jnp.where(ri < 0.5, 23.0, jnp.where(ri < 1.5, 16.0, 11.0))
    hs = jnp.where(ri < 0.5, 12.0, jnp.where(ri < 1.5, 16.0, 22.0))
    sc = jnp.where(si < 0.5, 8.0, jnp.where(si < 1.5, 16.0, 32.0))
    hw = 0.5 * (ws * sc - 1.0)
    hh = 0.5 * (hs * sc - 1.0)
    base = jnp.where(c < 0.5, 7.5 - hw,
                     jnp.where(c < 1.5, 7.5 - hh,
                               jnp.where(c < 2.5, 7.5 + hw, 7.5 + hh)))
    y = jnp.floor(k * (1.0 / 64.0))
    x = k - 64.0 * y
    c_even = jnp.logical_or(c < 0.5, jnp.abs(c - 2.0) < 0.5)
    return base + _FEAT_STRIDE * jnp.where(c_even, x, y)


def _body(scores_ref, bbox_ref, fg_ref, bbox_out_ref, anc_ref, pat_ref):
    @pl.when(pl.program_id(0) == 0)
    def _():
        pat_ref[...] = _anchor_pattern()

    fg_ref[...] = scores_ref[...]
    bbox_out_ref[...] = bbox_ref[...]
    anc_ref[0] = pat_ref[...]
    anc_ref[1] = pat_ref[...]


def kernel(scores, bbox_deltas, im_info, cfg_key):
    B = scores.shape[0]
    A = 9
    H, W = scores.shape[2], scores.shape[3]
    K = H * W

    C = bbox_deltas.shape[1]
    fg, bbox_out, anc = pl.pallas_call(
        _body,
        grid=(B // 2,),
        in_specs=[
            pl.BlockSpec((2, A, H, W), lambda b: (b, 1, 0, 0)),
            pl.BlockSpec((2, C, H, W), lambda b: (b, 0, 0, 0)),
        ],
        out_specs=[
            pl.BlockSpec((2, A, H, W), lambda b: (b, 0, 0, 0)),
            pl.BlockSpec((2, C, H, W), lambda b: (b, 0, 0, 0)),
            pl.BlockSpec((2, (K * A * 4) // 128, 128), lambda b: (b, 0, 0)),
        ],
        out_shape=[
            jax.ShapeDtypeStruct((B, A, H, W), jnp.float32),
            jax.ShapeDtypeStruct((B, C, H, W), jnp.float32),
            jax.ShapeDtypeStruct((B, (K * A * 4) // 128, 128), jnp.float32),
        ],
        scratch_shapes=[pltpu.VMEM(((K * A * 4) // 128, 128), jnp.float32)],
        compiler_params=pltpu.CompilerParams(
            dimension_semantics=("arbitrary",),
        ),
    )(scores, bbox_deltas)

    anchors = (anc.reshape(B, (K * A) // 128, 4, 128)
               .transpose(0, 1, 3, 2)
               .reshape(B, K * A, 4))
    return (fg, bbox_out, im_info, anchors)
